# Initial kernel scaffold; baseline (speedup 1.0000x reference)
import jax
import jax.numpy as jnp
from jax.experimental import pallas as pl


def kernel(input_ids, embedding_table):
    raise NotImplementedError("write your pallas kernel here")


# speedup vs baseline: 1.8579x; 1.8579x over previous
import functools

import jax
import jax.numpy as jnp
from jax import lax
from jax.experimental import pallas as pl
from jax.experimental.pallas import tpu as pltpu
from jax.experimental.pallas import tpu_sc as plsc

VOCAB = 1000000
EMB = 64
NC = 2
NS = 16
NW = NC * NS
IDX_W = 128
NBUF = 4


def _gather_kernel(n_chunks):
    per_w = n_chunks * IDX_W
    mesh = plsc.VectorSubcoreMesh(
        core_axis_name="c", subcore_axis_name="s", num_cores=NC, num_subcores=NS
    )

    @functools.partial(
        pl.kernel,
        out_type=jax.ShapeDtypeStruct((NW * per_w, EMB), jnp.float32),
        mesh=mesh,
        scratch_types=[
            pltpu.VMEM((n_chunks, IDX_W), jnp.int32),
            pltpu.VMEM((NBUF, IDX_W, EMB), jnp.float32),
            pltpu.SemaphoreType.DMA,
            pltpu.SemaphoreType.DMA,
        ],
        compiler_params=pltpu.CompilerParams(use_tc_tiling_on_sc=False),
    )
    def body(ids_hbm, table_hbm, out_hbm, idx_v, rows_v, gsem, ssem):
        wid = lax.axis_index("s") * NC + lax.axis_index("c")
        base = wid * per_w
        pltpu.sync_copy(ids_hbm.at[wid], idx_v)

        def group(g, _):
            j0 = g * NBUF
            for b in range(NBUF):
                pltpu.async_copy(
                    table_hbm.at[idx_v.at[j0 + b]], rows_v.at[b], gsem
                )
            for b in range(NBUF):
                pltpu.make_async_copy(
                    table_hbm.at[idx_v.at[j0 + b]], rows_v.at[b], gsem
                ).wait()
                pltpu.async_copy(
                    rows_v.at[b],
                    out_hbm.at[pl.ds(base + (j0 + b) * IDX_W, IDX_W)],
                    ssem,
                )
            for b in range(NBUF):
                pltpu.make_async_copy(
                    rows_v.at[b],
                    out_hbm.at[pl.ds(base + (j0 + b) * IDX_W, IDX_W)],
                    ssem,
                ).wait()
            return 0

        lax.fori_loop(0, n_chunks // NBUF, group, 0)

    return body


def kernel(input_ids, embedding_table):
    bsz, seq = input_ids.shape
    n_tok = bsz * seq
    n_chunks = n_tok // (NW * IDX_W)
    ids = input_ids.reshape(NW, n_chunks, IDX_W).astype(jnp.int32)
    out = _gather_kernel(n_chunks)(ids, embedding_table)
    return out.reshape(bsz, seq, EMB)


# speedup vs baseline: 1.8659x; 1.0043x over previous
import functools

import jax
import jax.numpy as jnp
from jax import lax
from jax.experimental import pallas as pl
from jax.experimental.pallas import tpu as pltpu
from jax.experimental.pallas import tpu_sc as plsc

VOCAB = 1000000
EMB = 64
NC = 2
NS = 16
NW = NC * NS
IDX_W = 128
NBUF = 8


def _gather_kernel(n_chunks):
    per_w = n_chunks * IDX_W
    mesh = plsc.VectorSubcoreMesh(
        core_axis_name="c", subcore_axis_name="s", num_cores=NC, num_subcores=NS
    )

    @functools.partial(
        pl.kernel,
        out_type=jax.ShapeDtypeStruct((NW * per_w, EMB), jnp.float32),
        mesh=mesh,
        scratch_types=[
            pltpu.VMEM((n_chunks, IDX_W), jnp.int32),
            pltpu.VMEM((NBUF, IDX_W, EMB), jnp.float32),
            pltpu.SemaphoreType.DMA,
            pltpu.SemaphoreType.DMA,
        ],
        compiler_params=pltpu.CompilerParams(use_tc_tiling_on_sc=False),
    )
    def body(ids_hbm, table_hbm, out_hbm, idx_v, rows_v, gsem, ssem):
        wid = lax.axis_index("s") * NC + lax.axis_index("c")
        base = wid * per_w
        pltpu.sync_copy(ids_hbm.at[wid], idx_v)

        def group(g, _):
            j0 = g * NBUF
            for b in range(NBUF):
                @pl.when(g > 0)
                def _():
                    pltpu.make_async_copy(
                        rows_v.at[b],
                        out_hbm.at[pl.ds(base + (j0 + b) * IDX_W, IDX_W)],
                        ssem,
                    ).wait()

                pltpu.async_copy(
                    table_hbm.at[idx_v.at[j0 + b]], rows_v.at[b], gsem
                )
            for b in range(NBUF):
                pltpu.make_async_copy(
                    table_hbm.at[idx_v.at[j0 + b]], rows_v.at[b], gsem
                ).wait()
                pltpu.async_copy(
                    rows_v.at[b],
                    out_hbm.at[pl.ds(base + (j0 + b) * IDX_W, IDX_W)],
                    ssem,
                )
            return 0

        n_groups = n_chunks // NBUF
        lax.fori_loop(0, n_groups, group, 0)
        for b in range(NBUF):
            pltpu.make_async_copy(
                rows_v.at[b],
                out_hbm.at[pl.ds(base + b * IDX_W, IDX_W)],
                ssem,
            ).wait()

    return body


def kernel(input_ids, embedding_table):
    bsz, seq = input_ids.shape
    n_tok = bsz * seq
    n_chunks = n_tok // (NW * IDX_W)
    ids = input_ids.reshape(NW, n_chunks, IDX_W).astype(jnp.int32)
    out = _gather_kernel(n_chunks)(ids, embedding_table)
    return out.reshape(bsz, seq, EMB)
